# baseline (speedup 1.0000x reference)
import functools

import jax
import jax.numpy as jnp
from jax import lax
from jax.experimental import pallas as pl
from jax.experimental.pallas import tpu as pltpu
from jax.experimental.pallas import tpu_sc as plsc

_N = 10000
_E = 320000
_D = 128
_C = 64
_K = 8
_ALPHA = 0.05

_NSUB = 16
_NCORE = 2
_CH = 128
_G = 3
_NCH = 162
_EPT = _NCH * _CH
_EP = _NSUB * _EPT
_RPT = 640
_NB = _RPT // _CH
_NR = _NSUB * _RPT
_CHALF = _C // _NCORE

_CKS = [(1.0 - _ALPHA) * (1.0 / _K) ** (_K + 1 - k) for k in range(1, _K + 1)]
_BETA = _ALPHA * sum((1.0 / _K) ** j for j in range(1, _K + 1))


def _project_body(f_ref, w_ref, o_ref):
    o_ref[...] = lax.dot_general(
        f_ref[...], w_ref[...],
        dimension_numbers=(((1,), (1,)), ((), ())),
        preferred_element_type=jnp.float32,
    )


def _propagate_body(y0_hbm, src_hbm, dst_hbm, b_hbm, out_hbm,
                    yA, yB, si, di, b0, b1, b2, b3, b4, b5,
                    acc, pidx, bv,
                    ga0, ga1, ga2, gb0, gb1, gb2,
                    sa0, sa1, sa2, sb0, sb1, sb2):
    c = lax.axis_index("c")
    s = lax.axis_index("s")
    row0 = s * _RPT
    bufA = (b0, b1, b2)
    bufB = (b3, b4, b5)
    gsA = (ga0, ga1, ga2)
    gsB = (gb0, gb1, gb2)
    ssA = (sa0, sa1, sa2)
    ssB = (sb0, sb1, sb2)

    pltpu.sync_copy(src_hbm.at[s], si)
    pltpu.sync_copy(dst_hbm.at[s], di)
    pltpu.sync_copy(b_hbm.at[pl.ds(c * _CHALF, _CHALF)], bv)

    lanes = lax.iota(jnp.int32, 16)
    for v in range(_CH // 16):
        pidx[0, pl.ds(v * 16, 16)] = lanes + (_N + v * 16)

    zv = jnp.zeros((16,), jnp.float32)

    def _zero_b0(i, carry):
        b0[i, pl.ds(0, 16)] = zv
        b0[i, pl.ds(16, 16)] = zv
        return carry

    pltpu.sync_copy(y0_hbm.at[c, pl.ds(row0, _RPT)], acc)
    pltpu.sync_copy(acc, yA.at[pl.ds(row0, _RPT)])
    lax.fori_loop(0, _CH, _zero_b0, 0)

    def _zero_blk_init(j, carry):
        pltpu.sync_copy(b0, yB.at[pl.ds(row0 + j * _CH, _CH)])
        return carry

    lax.fori_loop(0, _NB, _zero_blk_init, 0)
    plsc.subcore_barrier()

    for k in range(1, _K + 1):
        src_tab, dst_tab = (yA, yB) if k % 2 == 1 else (yB, yA)

        gdA = []
        sdB = []
        for j in range(_G):
            gdA.append(pltpu.async_copy(
                src_tab.at[si.at[j]], bufA[j], gsA[j]))
            sdB.append(pltpu.async_copy(
                bufB[j], dst_tab.at[pidx.at[0]], ssB[j], add=True))

        def _body(u, carry, src_tab=src_tab, dst_tab=dst_tab):
            base = u * 2 * _G
            for j in range(_G):
                pltpu.make_async_copy(
                    src_tab.at[si.at[base + j]], bufA[j], gsA[j]).wait()
                pltpu.async_copy(
                    bufA[j], dst_tab.at[di.at[base + j]], ssA[j], add=True)
            for j in range(_G):
                pltpu.make_async_copy(
                    bufB[j], dst_tab.at[pidx.at[0]], ssB[j]).wait()
                pltpu.async_copy(
                    src_tab.at[si.at[base + _G + j]], bufB[j], gsB[j])
            for j in range(_G):
                pltpu.make_async_copy(
                    src_tab.at[si.at[base + _G + j]], bufB[j], gsB[j]).wait()
                pltpu.async_copy(
                    bufB[j], dst_tab.at[di.at[base + _G + j]], ssB[j],
                    add=True)
            for j in range(_G):
                pltpu.make_async_copy(
                    bufA[j], dst_tab.at[pidx.at[0]], ssA[j]).wait()
                nxt = jnp.minimum(base + 2 * _G + j, _NCH - _G + j)
                pltpu.async_copy(src_tab.at[si.at[nxt]], bufA[j], gsA[j])
            return carry

        lax.fori_loop(0, _NCH // (2 * _G), _body, 0)

        for j in range(_G):
            pltpu.make_async_copy(
                bufB[j], dst_tab.at[pidx.at[0]], ssB[j]).wait()
            pltpu.make_async_copy(
                src_tab.at[si.at[_NCH - _G + j]], bufA[j], gsA[j]).wait()
        plsc.subcore_barrier()

        ck = _CKS[k - 1]
        if k == _K:
            blo = bv[pl.ds(0, 16)]
            bhi = bv[pl.ds(16, 16)]
        else:
            lax.fori_loop(0, _CH, _zero_b0, 0)

        def _upd_blk(j, carry, src_tab=src_tab, dst_tab=dst_tab, k=k, ck=ck):
            blk0 = j * _CH
            pltpu.sync_copy(dst_tab.at[pl.ds(row0 + blk0, _CH)], b1)

            if k == 1:
                def _fma(i, c2):
                    r = blk0 + i
                    for h in (0, 16):
                        acc[r, pl.ds(h, 16)] = (acc[r, pl.ds(h, 16)] * _BETA
                                                + b1[i, pl.ds(h, 16)] * ck)
                    return c2
            elif k < _K:
                def _fma(i, c2):
                    r = blk0 + i
                    for h in (0, 16):
                        acc[r, pl.ds(h, 16)] = (acc[r, pl.ds(h, 16)]
                                                + b1[i, pl.ds(h, 16)] * ck)
                    return c2
            else:
                def _fma(i, c2):
                    r = blk0 + i
                    acc[r, pl.ds(0, 16)] = (acc[r, pl.ds(0, 16)]
                                            + b1[i, pl.ds(0, 16)] * ck + blo)
                    acc[r, pl.ds(16, 16)] = (acc[r, pl.ds(16, 16)]
                                             + b1[i, pl.ds(16, 16)] * ck
                                             + bhi)
                    return c2

            lax.fori_loop(0, _CH, _fma, carry)
            if k < _K:
                pltpu.sync_copy(b0, src_tab.at[pl.ds(row0 + blk0, _CH)])
            return carry

        lax.fori_loop(0, _NB, _upd_blk, 0)
        if k < _K:
            plsc.subcore_barrier()

    pltpu.sync_copy(acc, out_hbm.at[c, s])


_propagate = functools.partial(
    pl.kernel,
    out_type=jax.ShapeDtypeStruct((_NCORE, _NSUB, _RPT, _CHALF), jnp.float32),
    mesh=plsc.VectorSubcoreMesh(
        core_axis_name="c", subcore_axis_name="s",
        num_cores=_NCORE, num_subcores=_NSUB),
    compiler_params=pltpu.CompilerParams(use_tc_tiling_on_sc=False),
    scratch_types=[
        pltpu.VMEM_SHARED((_NR, _CHALF), jnp.float32),
        pltpu.VMEM_SHARED((_NR, _CHALF), jnp.float32),
        pltpu.VMEM((_NCH, _CH), jnp.int32),
        pltpu.VMEM((_NCH, _CH), jnp.int32),
        pltpu.VMEM((_CH, _CHALF), jnp.float32),
        pltpu.VMEM((_CH, _CHALF), jnp.float32),
        pltpu.VMEM((_CH, _CHALF), jnp.float32),
        pltpu.VMEM((_CH, _CHALF), jnp.float32),
        pltpu.VMEM((_CH, _CHALF), jnp.float32),
        pltpu.VMEM((_CH, _CHALF), jnp.float32),
        pltpu.VMEM((_RPT, _CHALF), jnp.float32),
        pltpu.VMEM((1, _CH), jnp.int32),
        pltpu.VMEM((_CHALF,), jnp.float32),
        pltpu.SemaphoreType.DMA, pltpu.SemaphoreType.DMA,
        pltpu.SemaphoreType.DMA, pltpu.SemaphoreType.DMA,
        pltpu.SemaphoreType.DMA, pltpu.SemaphoreType.DMA,
        pltpu.SemaphoreType.DMA, pltpu.SemaphoreType.DMA,
        pltpu.SemaphoreType.DMA, pltpu.SemaphoreType.DMA,
        pltpu.SemaphoreType.DMA, pltpu.SemaphoreType.DMA,
    ],
)(_propagate_body)


def kernel(feat, edge_index, W, b):
    feat_p = jnp.pad(feat, ((0, _NR - _N), (0, 0)))
    y0 = pl.pallas_call(
        _project_body,
        out_shape=jax.ShapeDtypeStruct((_NR, _C), jnp.float32),
    )(feat_p, W)
    y0s = y0.reshape(_NR, _NCORE, _CHALF).transpose(1, 0, 2)

    src = edge_index[0]
    dst = edge_index[1]
    pad_idx = (_N + (jnp.arange(_EP - _E, dtype=jnp.int32) % (_NR - _N)))
    srcs = jnp.concatenate([src, pad_idx]).reshape(_NSUB, _NCH, _CH)
    dsts = jnp.concatenate([dst, pad_idx]).reshape(_NSUB, _NCH, _CH)

    out_sc = _propagate(y0s, srcs, dsts, b)
    return out_sc.transpose(1, 2, 0, 3).reshape(_NR, _C)[:_N]


# speedup vs baseline: 1.0793x; 1.0793x over previous
import functools

import jax
import jax.numpy as jnp
from jax import lax
from jax.experimental import pallas as pl
from jax.experimental.pallas import tpu as pltpu
from jax.experimental.pallas import tpu_sc as plsc

_N = 10000
_E = 320000
_D = 128
_C = 64
_K = 8
_ALPHA = 0.05

_NSUB = 16
_NCORE = 2
_CH = 128
_NCH = 160
_EPT = _NCH * _CH
_EP = _NSUB * _EPT
_RPT = 640
_NB = _RPT // _CH
_NR = _NSUB * _RPT
_CHALF = _C // _NCORE

_CKS = [(1.0 - _ALPHA) * (1.0 / _K) ** (_K + 1 - k) for k in range(1, _K + 1)]
_BETA = _ALPHA * sum((1.0 / _K) ** j for j in range(1, _K + 1))


def _project_body(f_ref, w_ref, o_ref):
    o_ref[...] = lax.dot_general(
        f_ref[...], w_ref[...],
        dimension_numbers=(((1,), (1,)), ((), ())),
        preferred_element_type=jnp.float32,
    )


def _propagate_body(y0_hbm, src_hbm, dst_hbm, b_hbm, out_hbm,
                    stab, htab, si, di, b0, b1, b2, b3,
                    acc, bv,
                    sg0, sg1, sg2, sg3, ss0, ss1, ss2, ss3):
    c = lax.axis_index("c")
    s = lax.axis_index("s")
    row0 = s * _RPT
    gbufs = (b0, b1, b2, b3)
    gsems = (sg0, sg1, sg2, sg3)
    ssems = (ss0, ss1, ss2, ss3)

    pltpu.sync_copy(src_hbm.at[s], si)
    pltpu.sync_copy(dst_hbm.at[s], di)
    pltpu.sync_copy(b_hbm.at[pl.ds(c * _CHALF, _CHALF)], bv)

    zv = jnp.zeros((16,), jnp.float32)

    def _zero_b0(i, carry):
        b0[i, pl.ds(0, 16)] = zv
        b0[i, pl.ds(16, 16)] = zv
        return carry

    pltpu.sync_copy(y0_hbm.at[c, pl.ds(row0, _RPT)], acc)
    lax.fori_loop(0, _CH, _zero_b0, 0)

    def _zero_blk_init(j, carry):
        pltpu.sync_copy(b0, stab.at[pl.ds(row0 + j * _CH, _CH)])
        return carry

    lax.fori_loop(0, _NB, _zero_blk_init, 0)
    plsc.subcore_barrier()

    for k in range(1, _K + 1):
        gtab = y0_hbm.at[c] if k == 1 else htab.at[c]

        def _edges(t, carry, gtab=gtab):
            base = t * 4
            gds = []
            for j in range(4):
                gds.append(pltpu.async_copy(
                    gtab.at[si.at[base + j]], gbufs[j], gsems[j]))
            sds = []
            for j in range(4):
                gds[j].wait()
                sds.append(pltpu.async_copy(
                    gbufs[j], stab.at[di.at[base + j]], ssems[j],
                    add=True))
            for sd in sds:
                sd.wait()
            return carry

        lax.fori_loop(0, _NCH // 4, _edges, 0)
        plsc.subcore_barrier()

        ck = _CKS[k - 1]
        if k == _K:
            blo = bv[pl.ds(0, 16)]
            bhi = bv[pl.ds(16, 16)]
        else:
            lax.fori_loop(0, _CH, _zero_b0, 0)

        def _upd_blk(j, carry, k=k, ck=ck):
            blk0 = j * _CH
            pltpu.sync_copy(stab.at[pl.ds(row0 + blk0, _CH)], b1)

            if k == 1:
                def _fma(i, c2):
                    r = blk0 + i
                    for h in (0, 16):
                        acc[r, pl.ds(h, 16)] = (acc[r, pl.ds(h, 16)] * _BETA
                                                + b1[i, pl.ds(h, 16)] * ck)
                    return c2
            elif k < _K:
                def _fma(i, c2):
                    r = blk0 + i
                    for h in (0, 16):
                        acc[r, pl.ds(h, 16)] = (acc[r, pl.ds(h, 16)]
                                                + b1[i, pl.ds(h, 16)] * ck)
                    return c2
            else:
                def _fma(i, c2):
                    r = blk0 + i
                    acc[r, pl.ds(0, 16)] = (acc[r, pl.ds(0, 16)]
                                            + b1[i, pl.ds(0, 16)] * ck + blo)
                    acc[r, pl.ds(16, 16)] = (acc[r, pl.ds(16, 16)]
                                             + b1[i, pl.ds(16, 16)] * ck
                                             + bhi)
                    return c2

            lax.fori_loop(0, _CH, _fma, carry)
            if k < _K:
                pltpu.sync_copy(b1, htab.at[c, pl.ds(row0 + blk0, _CH)])
                pltpu.sync_copy(b0, stab.at[pl.ds(row0 + blk0, _CH)])
            return carry

        lax.fori_loop(0, _NB, _upd_blk, 0)
        if k < _K:
            plsc.subcore_barrier()

    pltpu.sync_copy(acc, out_hbm.at[c, s])


_propagate = functools.partial(
    pl.kernel,
    out_type=jax.ShapeDtypeStruct((_NCORE, _NSUB, _RPT, _CHALF), jnp.float32),
    mesh=plsc.VectorSubcoreMesh(
        core_axis_name="c", subcore_axis_name="s",
        num_cores=_NCORE, num_subcores=_NSUB),
    compiler_params=pltpu.CompilerParams(use_tc_tiling_on_sc=False),
    scratch_types=[
        pltpu.VMEM_SHARED((_NR, _CHALF), jnp.float32),
        pltpu.HBM((_NCORE, _NR, _CHALF), jnp.float32),
        pltpu.VMEM((_NCH, _CH), jnp.int32),
        pltpu.VMEM((_NCH, _CH), jnp.int32),
        pltpu.VMEM((_CH, _CHALF), jnp.float32),
        pltpu.VMEM((_CH, _CHALF), jnp.float32),
        pltpu.VMEM((_CH, _CHALF), jnp.float32),
        pltpu.VMEM((_CH, _CHALF), jnp.float32),
        pltpu.VMEM((_RPT, _CHALF), jnp.float32),
        pltpu.VMEM((_CHALF,), jnp.float32),
        pltpu.SemaphoreType.DMA, pltpu.SemaphoreType.DMA,
        pltpu.SemaphoreType.DMA, pltpu.SemaphoreType.DMA,
        pltpu.SemaphoreType.DMA, pltpu.SemaphoreType.DMA,
        pltpu.SemaphoreType.DMA, pltpu.SemaphoreType.DMA,
    ],
)(_propagate_body)


def kernel(feat, edge_index, W, b):
    feat_p = jnp.pad(feat, ((0, _NR - _N), (0, 0)))
    y0 = pl.pallas_call(
        _project_body,
        out_shape=jax.ShapeDtypeStruct((_NR, _C), jnp.float32),
    )(feat_p, W)
    y0s = y0.reshape(_NR, _NCORE, _CHALF).transpose(1, 0, 2)

    src = edge_index[0]
    dst = edge_index[1]
    pad_idx = (_N + (jnp.arange(_EP - _E, dtype=jnp.int32) % (_NR - _N)))
    srcs = jnp.concatenate([src, pad_idx]).reshape(_NSUB, _NCH, _CH)
    dsts = jnp.concatenate([dst, pad_idx]).reshape(_NSUB, _NCH, _CH)

    out_sc = _propagate(y0s, srcs, dsts, b)
    return out_sc.transpose(1, 2, 0, 3).reshape(_NR, _C)[:_N]


# speedup vs baseline: 1.2750x; 1.1814x over previous
import functools

import jax
import jax.numpy as jnp
from jax import lax
from jax.experimental import pallas as pl
from jax.experimental.pallas import tpu as pltpu
from jax.experimental.pallas import tpu_sc as plsc

_N = 10000
_E = 320000
_D = 128
_C = 64
_K = 8
_ALPHA = 0.05

_NSUB = 16
_NCORE = 2
_CH = 128
_NF = 6
_NCH = 162
_EPT = _NCH * _CH
_EP = _NSUB * _EPT
_RPT = 640
_NB = _RPT // _CH
_NR = _NSUB * _RPT
_CHALF = _C // _NCORE

_CKS = [(1.0 - _ALPHA) * (1.0 / _K) ** (_K + 1 - k) for k in range(1, _K + 1)]
_BETA = _ALPHA * sum((1.0 / _K) ** j for j in range(1, _K + 1))


def _project_body(f_ref, w_ref, o_ref):
    o_ref[...] = lax.dot_general(
        f_ref[...], w_ref[...],
        dimension_numbers=(((1,), (1,)), ((), ())),
        preferred_element_type=jnp.float32,
    )


def _propagate_body(y0_hbm, src_hbm, dst_hbm, b_hbm, out_hbm,
                    yA, yB, si, di, b0, b1, b2, b3, b4, b5,
                    acc, bv,
                    sg0, sg1, sg2, sg3, sg4, sg5,
                    ss0, ss1, ss2, ss3, ss4, ss5):
    c = lax.axis_index("c")
    s = lax.axis_index("s")
    row0 = s * _RPT
    gbufs = (b0, b1, b2, b3, b4, b5)
    gsems = (sg0, sg1, sg2, sg3, sg4, sg5)
    ssems = (ss0, ss1, ss2, ss3, ss4, ss5)

    pltpu.sync_copy(src_hbm.at[s], si)
    pltpu.sync_copy(dst_hbm.at[s], di)
    pltpu.sync_copy(b_hbm.at[pl.ds(c * _CHALF, _CHALF)], bv)

    zv = jnp.zeros((16,), jnp.float32)

    def _zero_b0(i, carry):
        b0[i, pl.ds(0, 16)] = zv
        b0[i, pl.ds(16, 16)] = zv
        return carry

    pltpu.sync_copy(y0_hbm.at[c, pl.ds(row0, _RPT)], acc)
    pltpu.sync_copy(acc, yA.at[pl.ds(row0, _RPT)])
    lax.fori_loop(0, _CH, _zero_b0, 0)

    def _zero_blk_init(j, carry):
        pltpu.sync_copy(b0, yB.at[pl.ds(row0 + j * _CH, _CH)])
        return carry

    lax.fori_loop(0, _NB, _zero_blk_init, 0)
    plsc.subcore_barrier()

    for k in range(1, _K + 1):
        src_tab, dst_tab = (yA, yB) if k % 2 == 1 else (yB, yA)

        def _edges(t, carry, src_tab=src_tab, dst_tab=dst_tab):
            base = t * _NF
            gds = []
            for j in range(_NF):
                gds.append(pltpu.async_copy(
                    src_tab.at[si.at[base + j]], gbufs[j], gsems[j]))
            sds = []
            for j in range(_NF):
                gds[j].wait()
                sds.append(pltpu.async_copy(
                    gbufs[j], dst_tab.at[di.at[base + j]], ssems[j],
                    add=True))
            for sd in sds:
                sd.wait()
            return carry

        lax.fori_loop(0, _NCH // _NF, _edges, 0)
        plsc.subcore_barrier()

        ck = _CKS[k - 1]
        if k == _K:
            blo = bv[pl.ds(0, 16)]
            bhi = bv[pl.ds(16, 16)]
        else:
            lax.fori_loop(0, _CH, _zero_b0, 0)

        def _upd_blk(j, carry, src_tab=src_tab, dst_tab=dst_tab, k=k, ck=ck):
            blk0 = j * _CH
            pltpu.sync_copy(dst_tab.at[pl.ds(row0 + blk0, _CH)], b1)

            if k == 1:
                def _fma(i, c2):
                    r = blk0 + i
                    for h in (0, 16):
                        acc[r, pl.ds(h, 16)] = (acc[r, pl.ds(h, 16)] * _BETA
                                                + b1[i, pl.ds(h, 16)] * ck)
                    return c2
            elif k < _K:
                def _fma(i, c2):
                    r = blk0 + i
                    for h in (0, 16):
                        acc[r, pl.ds(h, 16)] = (acc[r, pl.ds(h, 16)]
                                                + b1[i, pl.ds(h, 16)] * ck)
                    return c2
            else:
                def _fma(i, c2):
                    r = blk0 + i
                    acc[r, pl.ds(0, 16)] = (acc[r, pl.ds(0, 16)]
                                            + b1[i, pl.ds(0, 16)] * ck + blo)
                    acc[r, pl.ds(16, 16)] = (acc[r, pl.ds(16, 16)]
                                             + b1[i, pl.ds(16, 16)] * ck
                                             + bhi)
                    return c2

            lax.fori_loop(0, _CH, _fma, carry)
            if k < _K:
                pltpu.sync_copy(b0, src_tab.at[pl.ds(row0 + blk0, _CH)])
            return carry

        lax.fori_loop(0, _NB, _upd_blk, 0)
        if k < _K:
            plsc.subcore_barrier()

    pltpu.sync_copy(acc, out_hbm.at[c, s])


_propagate = functools.partial(
    pl.kernel,
    out_type=jax.ShapeDtypeStruct((_NCORE, _NSUB, _RPT, _CHALF), jnp.float32),
    mesh=plsc.VectorSubcoreMesh(
        core_axis_name="c", subcore_axis_name="s",
        num_cores=_NCORE, num_subcores=_NSUB),
    compiler_params=pltpu.CompilerParams(use_tc_tiling_on_sc=False),
    scratch_types=[
        pltpu.VMEM_SHARED((_NR, _CHALF), jnp.float32),
        pltpu.VMEM_SHARED((_NR, _CHALF), jnp.float32),
        pltpu.VMEM((_NCH, _CH), jnp.int32),
        pltpu.VMEM((_NCH, _CH), jnp.int32),
        pltpu.VMEM((_CH, _CHALF), jnp.float32),
        pltpu.VMEM((_CH, _CHALF), jnp.float32),
        pltpu.VMEM((_CH, _CHALF), jnp.float32),
        pltpu.VMEM((_CH, _CHALF), jnp.float32),
        pltpu.VMEM((_CH, _CHALF), jnp.float32),
        pltpu.VMEM((_CH, _CHALF), jnp.float32),
        pltpu.VMEM((_RPT, _CHALF), jnp.float32),
        pltpu.VMEM((_CHALF,), jnp.float32),
        pltpu.SemaphoreType.DMA, pltpu.SemaphoreType.DMA,
        pltpu.SemaphoreType.DMA, pltpu.SemaphoreType.DMA,
        pltpu.SemaphoreType.DMA, pltpu.SemaphoreType.DMA,
        pltpu.SemaphoreType.DMA, pltpu.SemaphoreType.DMA,
        pltpu.SemaphoreType.DMA, pltpu.SemaphoreType.DMA,
        pltpu.SemaphoreType.DMA, pltpu.SemaphoreType.DMA,
    ],
)(_propagate_body)


def kernel(feat, edge_index, W, b):
    feat_p = jnp.pad(feat, ((0, _NR - _N), (0, 0)))
    y0 = pl.pallas_call(
        _project_body,
        out_shape=jax.ShapeDtypeStruct((_NR, _C), jnp.float32),
    )(feat_p, W)
    y0s = y0.reshape(_NR, _NCORE, _CHALF).transpose(1, 0, 2)

    src = edge_index[0]
    dst = edge_index[1]
    pad_idx = (_N + (jnp.arange(_EP - _E, dtype=jnp.int32) % (_NR - _N)))
    srcs = jnp.concatenate([src, pad_idx]).reshape(_NSUB, _NCH, _CH)
    dsts = jnp.concatenate([dst, pad_idx]).reshape(_NSUB, _NCH, _CH)

    out_sc = _propagate(y0s, srcs, dsts, b)
    return out_sc.transpose(1, 2, 0, 3).reshape(_NR, _C)[:_N]
